# baseline (speedup 1.0000x reference)
import functools

import jax
import jax.numpy as jnp
from jax import lax
from jax.experimental import pallas as pl
from jax.experimental.pallas import tpu as pltpu
from jax.experimental.pallas import tpu_sc as plsc

N = 10000
D = 128
E = 320000

NC = 2
NS = 16
NW = NC * NS
NE_T = E // NW
G = 80
NCH = NE_T // G
RT = 624
REM0 = NS * RT
REM = N - REM0
ZCH = 208

_mesh = plsc.VectorSubcoreMesh(core_axis_name="c", subcore_axis_name="s")

_f32 = jnp.float32


def _zero_vmem_2d(buf, rows, cols):
    zf = jnp.zeros((16,), _f32)

    def row_body(r, _):
        def col_body(k, _):
            buf[r, pl.ds(k * 16, 16)] = zf
            return 0
        return lax.fori_loop(0, cols // 16, col_body, 0)

    lax.fori_loop(0, rows, row_body, 0)


def _sc_sum_body(x_hbm, src_hbm, dst_hbm, sums_out,
                 src_v0, dst_v0, rows_v0, src_v1, dst_v1, rows_v1,
                 zbuf, acc_sh, sem_g0, sem_g1, sem_s0, sem_s1):
    cid = lax.axis_index("c")
    sid = lax.axis_index("s")
    wid = cid * NS + sid

    _zero_vmem_2d(zbuf, ZCH, 128)
    row0 = sid * RT
    is_last = sid == NS - 1

    def zcopy(j, _):
        pltpu.sync_copy(zbuf, acc_sh.at[pl.ds(row0 + j * ZCH, ZCH)])
        return 0
    lax.fori_loop(0, RT // ZCH, zcopy, 0)

    @pl.when(is_last)
    def _():
        pltpu.sync_copy(zbuf.at[pl.ds(0, REM)], acc_sh.at[pl.ds(REM0, REM)])

    plsc.subcore_barrier()

    base_e = wid * NE_T
    bufs = ((src_v0, dst_v0, rows_v0, sem_g0, sem_s0),
            (src_v1, dst_v1, rows_v1, sem_g1, sem_s1))

    sv, dv, rv, sg, ss = bufs[0]
    sv1, dv1, rv1, sg1, ss1 = bufs[1]
    pltpu.sync_copy(src_hbm.at[pl.ds(base_e, G)], sv)
    pltpu.sync_copy(dst_hbm.at[pl.ds(base_e, G)], dv)
    pltpu.async_copy(x_hbm.at[sv], rv, sg)
    pltpu.make_async_copy(x_hbm.at[sv], rv, sg).wait()
    pltpu.async_copy(rv, acc_sh.at[dv], ss, add=True)
    pltpu.sync_copy(src_hbm.at[pl.ds(base_e + G, G)], sv1)
    pltpu.sync_copy(dst_hbm.at[pl.ds(base_e + G, G)], dv1)
    pltpu.async_copy(x_hbm.at[sv1], rv1, sg1)

    def pbody(p, _):
        for k in (1, 2):
            b = k % 2
            sv, dv, rv, sg, ss = bufs[b]
            svp, dvp, rvp, sgp, ssp = bufs[b ^ 1]
            pltpu.make_async_copy(x_hbm.at[sv], rv, sg).wait()
            pltpu.async_copy(rv, acc_sh.at[dv], ss, add=True)
            pltpu.make_async_copy(rvp, acc_sh.at[dvp], ssp).wait()
            cn = jnp.minimum(2 * p + k + 1, NCH - 1)
            off = base_e + cn * G
            pltpu.sync_copy(src_hbm.at[pl.ds(off, G)], svp)
            pltpu.sync_copy(dst_hbm.at[pl.ds(off, G)], dvp)
            pltpu.async_copy(x_hbm.at[svp], rvp, sgp)
        return 0
    lax.fori_loop(0, (NCH - 1) // 2, pbody, 0)
    pltpu.make_async_copy(rows_v0, acc_sh.at[dst_v0], sem_s0).wait()
    pltpu.make_async_copy(x_hbm.at[src_v1], rows_v1, sem_g1).wait()

    plsc.subcore_barrier()

    out_base = cid * N + row0
    pltpu.sync_copy(acc_sh.at[pl.ds(row0, RT)],
                    sums_out.at[pl.ds(out_base, RT)])

    @pl.when(is_last)
    def _():
        pltpu.sync_copy(acc_sh.at[pl.ds(REM0, REM)],
                        sums_out.at[pl.ds(cid * N + REM0, REM)])


def _sc_cnt_body(dst_hbm, cnts_out, dst_v, ones_v, zbuf, cnt_sh, sem):
    cid = lax.axis_index("c")
    sid = lax.axis_index("s")
    wid = cid * NS + sid

    _zero_vmem_2d(zbuf, ZCH, 128)
    row0 = sid * RT
    is_last = sid == NS - 1

    def zcopy(j, _):
        pltpu.sync_copy(zbuf, cnt_sh.at[pl.ds(row0 + j * ZCH, ZCH)])
        return 0
    lax.fori_loop(0, RT // ZCH, zcopy, 0)

    @pl.when(is_last)
    def _():
        pltpu.sync_copy(zbuf.at[pl.ds(0, REM)], cnt_sh.at[pl.ds(REM0, REM)])

    one = jnp.ones((16,), _f32)

    def ofill(i, _):
        def ocol(k, _):
            ones_v[i, pl.ds(k * 16, 16)] = one
            return 0
        return lax.fori_loop(0, 128 // 16, ocol, 0)
    lax.fori_loop(0, G, ofill, 0)

    plsc.subcore_barrier()

    base_e = wid * NE_T

    def ebody(i, _):
        off = base_e + i * G
        pltpu.sync_copy(dst_hbm.at[pl.ds(off, G)], dst_v)
        pltpu.sync_copy(ones_v, cnt_sh.at[dst_v], add=True)
        return 0
    lax.fori_loop(0, NCH, ebody, 0)

    plsc.subcore_barrier()

    out_base = cid * N + row0
    pltpu.sync_copy(cnt_sh.at[pl.ds(row0, RT)],
                    cnts_out.at[pl.ds(out_base, RT)])

    @pl.when(is_last)
    def _():
        pltpu.sync_copy(cnt_sh.at[pl.ds(REM0, REM)],
                        cnts_out.at[pl.ds(cid * N + REM0, REM)])


_sc_layer = pl.kernel(
    _sc_sum_body,
    out_type=jax.ShapeDtypeStruct((2 * N, 128), _f32),
    mesh=_mesh,
    scratch_types=[
        pltpu.VMEM((G,), jnp.int32),
        pltpu.VMEM((G,), jnp.int32),
        pltpu.VMEM((G, 128), _f32),
        pltpu.VMEM((G,), jnp.int32),
        pltpu.VMEM((G,), jnp.int32),
        pltpu.VMEM((G, 128), _f32),
        pltpu.VMEM((ZCH, 128), _f32),
        pltpu.VMEM_SHARED((N, 128), _f32),
        pltpu.SemaphoreType.DMA,
        pltpu.SemaphoreType.DMA,
        pltpu.SemaphoreType.DMA,
        pltpu.SemaphoreType.DMA,
    ],
)

_sc_counts = pl.kernel(
    _sc_cnt_body,
    out_type=jax.ShapeDtypeStruct((2 * N, 128), _f32),
    mesh=_mesh,
    scratch_types=[
        pltpu.VMEM((G,), jnp.int32),
        pltpu.VMEM((G, 128), _f32),
        pltpu.VMEM((ZCH, 128), _f32),
        pltpu.VMEM_SHARED((N, 128), _f32),
        pltpu.SemaphoreType.DMA,
    ],
)

BR = 400
NBLK = N // BR


def _tc_body(s0, s1, c0, c1, h, wl, wr, bl, out):
    s = s0[...] + s1[...]
    cnt = c0[:, :1] + c1[:, :1]
    inv = 1.0 / jnp.maximum(cnt, 1.0)
    z = (jnp.dot(s * inv, wl[...], preferred_element_type=_f32)
         + jnp.dot(h[...], wr[...], preferred_element_type=_f32)
         + bl[...])
    out[...] = jnp.where(z > 0, z, jnp.exp(z) - 1.0)


_tc_layer = pl.pallas_call(
    _tc_body,
    grid=(NBLK,),
    in_specs=[
        pl.BlockSpec((BR, 128), lambda i: (i, 0)),
        pl.BlockSpec((BR, 128), lambda i: (i + NBLK, 0)),
        pl.BlockSpec((BR, 8), lambda i: (i, 0)),
        pl.BlockSpec((BR, 8), lambda i: (i + NBLK, 0)),
        pl.BlockSpec((BR, 128), lambda i: (i, 0)),
        pl.BlockSpec((128, 128), lambda i: (0, 0)),
        pl.BlockSpec((128, 128), lambda i: (0, 0)),
        pl.BlockSpec((1, 128), lambda i: (0, 0)),
    ],
    out_specs=pl.BlockSpec((BR, 128), lambda i: (i, 0)),
    out_shape=jax.ShapeDtypeStruct((N, 128), _f32),
)


def kernel(x, edge_index, Wl1, Wr1, bl1, Wl2, Wr2, bl2, Wl3, Wr3, bl3):
    src = edge_index[0]
    dst = edge_index[1]

    cnts = _sc_counts(dst)[:, :8]
    sums1 = _sc_layer(x, src, dst)
    h1 = _tc_layer(sums1, sums1, cnts, cnts, x, Wl1, Wr1, bl1.reshape(1, 128))

    sums2 = _sc_layer(h1, src, dst)
    h2 = _tc_layer(sums2, sums2, cnts, cnts, h1, Wl2, Wr2, bl2.reshape(1, 128))

    sums3 = _sc_layer(h2, src, dst)
    h3 = _tc_layer(sums3, sums3, cnts, cnts, h2, Wl3, Wr3, bl3.reshape(1, 128))
    return h3


# speedup vs baseline: 1.0548x; 1.0548x over previous
import functools

import jax
import jax.numpy as jnp
from jax import lax
from jax.experimental import pallas as pl
from jax.experimental.pallas import tpu as pltpu
from jax.experimental.pallas import tpu_sc as plsc

N = 10000
D = 128
E = 320000

NC = 2
NS = 16
NW = NC * NS
NE_T = E // NW
G = 80
NCH = NE_T // G
RT = 624
REM0 = NS * RT
REM = N - REM0
ZCH = 208

_mesh = plsc.VectorSubcoreMesh(core_axis_name="c", subcore_axis_name="s")

_f32 = jnp.float32


def _zero_vmem_2d(buf, rows, cols):
    zf = jnp.zeros((16,), _f32)

    def row_body(r, _):
        def col_body(k, _):
            buf[r, pl.ds(k * 16, 16)] = zf
            return 0
        return lax.fori_loop(0, cols // 16, col_body, 0)

    lax.fori_loop(0, rows, row_body, 0)


def _sc_sum_body(x_hbm, src_hbm, dst_hbm, sums_out,
                 src_v0, dst_v0, rows_v0, src_v1, dst_v1, rows_v1,
                 zbuf, acc_sh, sem_g0, sem_g1, sem_s0, sem_s1):
    cid = lax.axis_index("c")
    sid = lax.axis_index("s")
    wid = cid * NS + sid

    _zero_vmem_2d(zbuf, ZCH, 128)
    row0 = sid * RT
    is_last = sid == NS - 1

    def zcopy(j, _):
        pltpu.sync_copy(zbuf, acc_sh.at[pl.ds(row0 + j * ZCH, ZCH)])
        return 0
    lax.fori_loop(0, RT // ZCH, zcopy, 0)

    @pl.when(is_last)
    def _():
        pltpu.sync_copy(zbuf.at[pl.ds(0, REM)], acc_sh.at[pl.ds(REM0, REM)])

    plsc.subcore_barrier()

    base_e = wid * NE_T
    bufs = ((src_v0, dst_v0, rows_v0, sem_g0, sem_s0),
            (src_v1, dst_v1, rows_v1, sem_g1, sem_s1))

    sv, dv, rv, sg, ss = bufs[0]
    sv1, dv1, rv1, sg1, ss1 = bufs[1]
    pltpu.sync_copy(src_hbm.at[pl.ds(base_e, G)], sv)
    pltpu.sync_copy(dst_hbm.at[pl.ds(base_e, G)], dv)
    pltpu.async_copy(x_hbm.at[sv], rv, sg)
    pltpu.make_async_copy(x_hbm.at[sv], rv, sg).wait()
    pltpu.async_copy(rv, acc_sh.at[dv], ss, add=True)
    pltpu.sync_copy(src_hbm.at[pl.ds(base_e + G, G)], sv1)
    pltpu.sync_copy(dst_hbm.at[pl.ds(base_e + G, G)], dv1)
    pltpu.async_copy(x_hbm.at[sv1], rv1, sg1)

    def pbody(p, _):
        for k in (1, 2):
            b = k % 2
            sv, dv, rv, sg, ss = bufs[b]
            svp, dvp, rvp, sgp, ssp = bufs[b ^ 1]
            pltpu.make_async_copy(x_hbm.at[sv], rv, sg).wait()
            pltpu.async_copy(rv, acc_sh.at[dv], ss, add=True)
            pltpu.make_async_copy(rvp, acc_sh.at[dvp], ssp).wait()
            cn = jnp.minimum(2 * p + k + 1, NCH - 1)
            off = base_e + cn * G
            pltpu.sync_copy(src_hbm.at[pl.ds(off, G)], svp)
            pltpu.sync_copy(dst_hbm.at[pl.ds(off, G)], dvp)
            pltpu.async_copy(x_hbm.at[svp], rvp, sgp)
        return 0
    lax.fori_loop(0, (NCH - 1) // 2, pbody, 0)
    pltpu.make_async_copy(rows_v0, acc_sh.at[dst_v0], sem_s0).wait()
    pltpu.make_async_copy(x_hbm.at[src_v1], rows_v1, sem_g1).wait()

    plsc.subcore_barrier()

    out_base = cid * N + row0
    pltpu.sync_copy(acc_sh.at[pl.ds(row0, RT)],
                    sums_out.at[pl.ds(out_base, RT)])

    @pl.when(is_last)
    def _():
        pltpu.sync_copy(acc_sh.at[pl.ds(REM0, REM)],
                        sums_out.at[pl.ds(cid * N + REM0, REM)])


def _sc_cnt_body(dst_hbm, cnts_out, dst_v0, dst_v1, ones_v, zbuf, cnt_sh,
                 sem_s0, sem_s1):
    cid = lax.axis_index("c")
    sid = lax.axis_index("s")
    wid = cid * NS + sid

    _zero_vmem_2d(zbuf, ZCH, 128)
    row0 = sid * RT
    is_last = sid == NS - 1

    def zcopy(j, _):
        pltpu.sync_copy(zbuf, cnt_sh.at[pl.ds(row0 + j * ZCH, ZCH)])
        return 0
    lax.fori_loop(0, RT // ZCH, zcopy, 0)

    @pl.when(is_last)
    def _():
        pltpu.sync_copy(zbuf.at[pl.ds(0, REM)], cnt_sh.at[pl.ds(REM0, REM)])

    one = jnp.ones((16,), _f32)

    def ofill(i, _):
        def ocol(k, _):
            ones_v[i, pl.ds(k * 16, 16)] = one
            return 0
        return lax.fori_loop(0, 128 // 16, ocol, 0)
    lax.fori_loop(0, G, ofill, 0)

    plsc.subcore_barrier()

    base_e = wid * NE_T
    dbufs = ((dst_v0, sem_s0), (dst_v1, sem_s1))

    pltpu.sync_copy(dst_hbm.at[pl.ds(base_e, G)], dst_v0)
    pltpu.async_copy(ones_v, cnt_sh.at[dst_v0], sem_s0, add=True)

    def ebody(p, _):
        for k in (1, 2):
            b = k % 2
            dv, ss = dbufs[b]
            dvp, ssp = dbufs[b ^ 1]
            off = base_e + (2 * p + k) * G
            pltpu.sync_copy(dst_hbm.at[pl.ds(off, G)], dv)
            pltpu.async_copy(ones_v, cnt_sh.at[dv], ss, add=True)
            pltpu.make_async_copy(ones_v, cnt_sh.at[dvp], ssp).wait()
        return 0
    lax.fori_loop(0, (NCH - 1) // 2, ebody, 0)
    pltpu.make_async_copy(ones_v, cnt_sh.at[dst_v0], sem_s0).wait()

    plsc.subcore_barrier()

    out_base = cid * N + row0
    pltpu.sync_copy(cnt_sh.at[pl.ds(row0, RT)],
                    cnts_out.at[pl.ds(out_base, RT)])

    @pl.when(is_last)
    def _():
        pltpu.sync_copy(cnt_sh.at[pl.ds(REM0, REM)],
                        cnts_out.at[pl.ds(cid * N + REM0, REM)])


_sc_layer = pl.kernel(
    _sc_sum_body,
    out_type=jax.ShapeDtypeStruct((2 * N, 128), _f32),
    mesh=_mesh,
    scratch_types=[
        pltpu.VMEM((G,), jnp.int32),
        pltpu.VMEM((G,), jnp.int32),
        pltpu.VMEM((G, 128), _f32),
        pltpu.VMEM((G,), jnp.int32),
        pltpu.VMEM((G,), jnp.int32),
        pltpu.VMEM((G, 128), _f32),
        pltpu.VMEM((ZCH, 128), _f32),
        pltpu.VMEM_SHARED((N, 128), _f32),
        pltpu.SemaphoreType.DMA,
        pltpu.SemaphoreType.DMA,
        pltpu.SemaphoreType.DMA,
        pltpu.SemaphoreType.DMA,
    ],
)

_sc_counts = pl.kernel(
    _sc_cnt_body,
    out_type=jax.ShapeDtypeStruct((2 * N, 128), _f32),
    mesh=_mesh,
    scratch_types=[
        pltpu.VMEM((G,), jnp.int32),
        pltpu.VMEM((G,), jnp.int32),
        pltpu.VMEM((G, 128), _f32),
        pltpu.VMEM((ZCH, 128), _f32),
        pltpu.VMEM_SHARED((N, 128), _f32),
        pltpu.SemaphoreType.DMA,
        pltpu.SemaphoreType.DMA,
    ],
)

BR = 400
NBLK = N // BR


def _tc_body(s0, s1, c0, c1, h, wl, wr, bl, out):
    s = s0[...] + s1[...]
    cnt = c0[:, :1] + c1[:, :1]
    inv = 1.0 / jnp.maximum(cnt, 1.0)
    z = (jnp.dot(s * inv, wl[...], preferred_element_type=_f32)
         + jnp.dot(h[...], wr[...], preferred_element_type=_f32)
         + bl[...])
    out[...] = jnp.where(z > 0, z, jnp.exp(z) - 1.0)


_tc_layer = pl.pallas_call(
    _tc_body,
    grid=(NBLK,),
    in_specs=[
        pl.BlockSpec((BR, 128), lambda i: (i, 0)),
        pl.BlockSpec((BR, 128), lambda i: (i + NBLK, 0)),
        pl.BlockSpec((BR, 8), lambda i: (i, 0)),
        pl.BlockSpec((BR, 8), lambda i: (i + NBLK, 0)),
        pl.BlockSpec((BR, 128), lambda i: (i, 0)),
        pl.BlockSpec((128, 128), lambda i: (0, 0)),
        pl.BlockSpec((128, 128), lambda i: (0, 0)),
        pl.BlockSpec((1, 128), lambda i: (0, 0)),
    ],
    out_specs=pl.BlockSpec((BR, 128), lambda i: (i, 0)),
    out_shape=jax.ShapeDtypeStruct((N, 128), _f32),
)


def kernel(x, edge_index, Wl1, Wr1, bl1, Wl2, Wr2, bl2, Wl3, Wr3, bl3):
    src = edge_index[0]
    dst = edge_index[1]

    cnts = _sc_counts(dst)[:, :8]
    sums1 = _sc_layer(x, src, dst)
    h1 = _tc_layer(sums1, sums1, cnts, cnts, x, Wl1, Wr1, bl1.reshape(1, 128))

    sums2 = _sc_layer(h1, src, dst)
    h2 = _tc_layer(sums2, sums2, cnts, cnts, h1, Wl2, Wr2, bl2.reshape(1, 128))

    sums3 = _sc_layer(h2, src, dst)
    h3 = _tc_layer(sums3, sums3, cnts, cnts, h2, Wl3, Wr3, bl3.reshape(1, 128))
    return h3


# speedup vs baseline: 1.2170x; 1.1537x over previous
import functools

import jax
import jax.numpy as jnp
from jax import lax
from jax.experimental import pallas as pl
from jax.experimental.pallas import tpu as pltpu
from jax.experimental.pallas import tpu_sc as plsc

N = 10000
D = 128
E = 320000

NC = 2
NS = 16
NW = NC * NS
NE_T = E // NW
G = 80
NCH = NE_T // G
RT = 624
REM0 = NS * RT
REM = N - REM0
ZCH = 208

_mesh = plsc.VectorSubcoreMesh(core_axis_name="c", subcore_axis_name="s")

_f32 = jnp.float32


def _zero_vmem_2d(buf, rows, cols):
    zf = jnp.zeros((16,), _f32)

    def row_body(r, _):
        def col_body(k, _):
            buf[r, pl.ds(k * 16, 16)] = zf
            return 0
        return lax.fori_loop(0, cols // 16, col_body, 0)

    lax.fori_loop(0, rows, row_body, 0)


def _sc_sum_body(x_hbm, ei3_hbm, sums_out,
                 iv0, rows_v0, iv1, rows_v1,
                 zbuf, acc_sh, sem_g0, sem_g1, sem_s0, sem_s1):
    cid = lax.axis_index("c")
    sid = lax.axis_index("s")
    wid = cid * NS + sid

    _zero_vmem_2d(zbuf, ZCH, 128)
    row0 = sid * RT
    is_last = sid == NS - 1

    def zcopy(j, _):
        pltpu.sync_copy(zbuf, acc_sh.at[pl.ds(row0 + j * ZCH, ZCH)])
        return 0
    lax.fori_loop(0, RT // ZCH, zcopy, 0)

    @pl.when(is_last)
    def _():
        pltpu.sync_copy(zbuf.at[pl.ds(0, REM)], acc_sh.at[pl.ds(REM0, REM)])

    plsc.subcore_barrier()

    base_c = wid * NCH
    bufs = ((iv0, rows_v0, sem_g0, sem_s0),
            (iv1, rows_v1, sem_g1, sem_s1))

    iv, rv, sg, ss = bufs[0]
    ivb, rv1, sg1, ss1 = bufs[1]
    pltpu.sync_copy(ei3_hbm.at[base_c], iv)
    pltpu.async_copy(x_hbm.at[iv.at[0]], rv, sg)
    pltpu.make_async_copy(x_hbm.at[iv.at[0]], rv, sg).wait()
    pltpu.async_copy(rv, acc_sh.at[iv.at[1]], ss, add=True)
    pltpu.sync_copy(ei3_hbm.at[base_c + 1], ivb)
    pltpu.async_copy(x_hbm.at[ivb.at[0]], rv1, sg1)

    def pbody(p, _):
        for k in (1, 2):
            b = k % 2
            iv, rv, sg, ss = bufs[b]
            ivp, rvp, sgp, ssp = bufs[b ^ 1]
            pltpu.make_async_copy(x_hbm.at[iv.at[0]], rv, sg).wait()
            pltpu.async_copy(rv, acc_sh.at[iv.at[1]], ss, add=True)
            pltpu.make_async_copy(rvp, acc_sh.at[ivp.at[1]], ssp).wait()
            cn = jnp.minimum(2 * p + k + 1, NCH - 1)
            pltpu.sync_copy(ei3_hbm.at[base_c + cn], ivp)
            pltpu.async_copy(x_hbm.at[ivp.at[0]], rvp, sgp)
        return 0
    lax.fori_loop(0, (NCH - 1) // 2, pbody, 0)
    pltpu.make_async_copy(rows_v0, acc_sh.at[iv0.at[1]], sem_s0).wait()
    pltpu.make_async_copy(x_hbm.at[iv1.at[0]], rows_v1, sem_g1).wait()

    plsc.subcore_barrier()

    out_base = cid * N + row0
    pltpu.sync_copy(acc_sh.at[pl.ds(row0, RT)],
                    sums_out.at[pl.ds(out_base, RT)])

    @pl.when(is_last)
    def _():
        pltpu.sync_copy(acc_sh.at[pl.ds(REM0, REM)],
                        sums_out.at[pl.ds(cid * N + REM0, REM)])


def _sc_cnt_body(dst_hbm, cnts_out, dst_v0, dst_v1, ones_v, zbuf, cnt_sh,
                 sem_s0, sem_s1):
    cid = lax.axis_index("c")
    sid = lax.axis_index("s")
    wid = cid * NS + sid

    _zero_vmem_2d(zbuf, ZCH, 128)
    row0 = sid * RT
    is_last = sid == NS - 1

    def zcopy(j, _):
        pltpu.sync_copy(zbuf, cnt_sh.at[pl.ds(row0 + j * ZCH, ZCH)])
        return 0
    lax.fori_loop(0, RT // ZCH, zcopy, 0)

    @pl.when(is_last)
    def _():
        pltpu.sync_copy(zbuf.at[pl.ds(0, REM)], cnt_sh.at[pl.ds(REM0, REM)])

    one = jnp.ones((16,), _f32)

    def ofill(i, _):
        def ocol(k, _):
            ones_v[i, pl.ds(k * 16, 16)] = one
            return 0
        return lax.fori_loop(0, 128 // 16, ocol, 0)
    lax.fori_loop(0, G, ofill, 0)

    plsc.subcore_barrier()

    base_e = wid * NE_T
    dbufs = ((dst_v0, sem_s0), (dst_v1, sem_s1))

    pltpu.sync_copy(dst_hbm.at[pl.ds(base_e, G)], dst_v0)
    pltpu.async_copy(ones_v, cnt_sh.at[dst_v0], sem_s0, add=True)

    def ebody(p, _):
        for k in (1, 2):
            b = k % 2
            dv, ss = dbufs[b]
            dvp, ssp = dbufs[b ^ 1]
            off = base_e + (2 * p + k) * G
            pltpu.sync_copy(dst_hbm.at[pl.ds(off, G)], dv)
            pltpu.async_copy(ones_v, cnt_sh.at[dv], ss, add=True)
            pltpu.make_async_copy(ones_v, cnt_sh.at[dvp], ssp).wait()
        return 0
    lax.fori_loop(0, (NCH - 1) // 2, ebody, 0)
    pltpu.make_async_copy(ones_v, cnt_sh.at[dst_v0], sem_s0).wait()

    plsc.subcore_barrier()

    out_base = cid * N + row0
    pltpu.sync_copy(cnt_sh.at[pl.ds(row0, RT)],
                    cnts_out.at[pl.ds(out_base, RT)])

    @pl.when(is_last)
    def _():
        pltpu.sync_copy(cnt_sh.at[pl.ds(REM0, REM)],
                        cnts_out.at[pl.ds(cid * N + REM0, REM)])


_sc_layer = pl.kernel(
    _sc_sum_body,
    out_type=jax.ShapeDtypeStruct((2 * N, 128), _f32),
    mesh=_mesh,
    scratch_types=[
        pltpu.VMEM((2, G), jnp.int32),
        pltpu.VMEM((G, 128), _f32),
        pltpu.VMEM((2, G), jnp.int32),
        pltpu.VMEM((G, 128), _f32),
        pltpu.VMEM((ZCH, 128), _f32),
        pltpu.VMEM_SHARED((N, 128), _f32),
        pltpu.SemaphoreType.DMA,
        pltpu.SemaphoreType.DMA,
        pltpu.SemaphoreType.DMA,
        pltpu.SemaphoreType.DMA,
    ],
)

_sc_counts = pl.kernel(
    _sc_cnt_body,
    out_type=jax.ShapeDtypeStruct((2 * N, 128), _f32),
    mesh=_mesh,
    scratch_types=[
        pltpu.VMEM((G,), jnp.int32),
        pltpu.VMEM((G,), jnp.int32),
        pltpu.VMEM((G, 128), _f32),
        pltpu.VMEM((ZCH, 128), _f32),
        pltpu.VMEM_SHARED((N, 128), _f32),
        pltpu.SemaphoreType.DMA,
        pltpu.SemaphoreType.DMA,
    ],
)

BR = 400
NBLK = N // BR


def _tc_body(s0, s1, c0, c1, h, wl, wr, bl, out):
    s = s0[...] + s1[...]
    cnt = c0[:, :1] + c1[:, :1]
    inv = 1.0 / jnp.maximum(cnt, 1.0)
    z = (jnp.dot(s * inv, wl[...], preferred_element_type=_f32)
         + jnp.dot(h[...], wr[...], preferred_element_type=_f32)
         + bl[...])
    out[...] = jnp.where(z > 0, z, jnp.exp(z) - 1.0)


_tc_layer = pl.pallas_call(
    _tc_body,
    grid=(NBLK,),
    in_specs=[
        pl.BlockSpec((BR, 128), lambda i: (i, 0)),
        pl.BlockSpec((BR, 128), lambda i: (i + NBLK, 0)),
        pl.BlockSpec((BR, 8), lambda i: (i, 0)),
        pl.BlockSpec((BR, 8), lambda i: (i + NBLK, 0)),
        pl.BlockSpec((BR, 128), lambda i: (i, 0)),
        pl.BlockSpec((128, 128), lambda i: (0, 0)),
        pl.BlockSpec((128, 128), lambda i: (0, 0)),
        pl.BlockSpec((1, 128), lambda i: (0, 0)),
    ],
    out_specs=pl.BlockSpec((BR, 128), lambda i: (i, 0)),
    out_shape=jax.ShapeDtypeStruct((N, 128), _f32),
)


def kernel(x, edge_index, Wl1, Wr1, bl1, Wl2, Wr2, bl2, Wl3, Wr3, bl3):
    dst = edge_index[1]
    ei3 = edge_index.reshape(2, NW * NCH, G).transpose(1, 0, 2)

    cnts = _sc_counts(dst)[:, :8]
    sums1 = _sc_layer(x, ei3)
    h1 = _tc_layer(sums1, sums1, cnts, cnts, x, Wl1, Wr1, bl1.reshape(1, 128))

    sums2 = _sc_layer(h1, ei3)
    h2 = _tc_layer(sums2, sums2, cnts, cnts, h1, Wl2, Wr2, bl2.reshape(1, 128))

    sums3 = _sc_layer(h2, ei3)
    h3 = _tc_layer(sums3, sums3, cnts, cnts, h2, Wl3, Wr3, bl3.reshape(1, 128))
    return h3
